# baseline (speedup 1.0000x reference)
import functools
import jax
import jax.numpy as jnp
from jax import lax
from jax.experimental import pallas as pl
from jax.experimental.pallas import tpu as pltpu
from jax.experimental.pallas import tpu_sc as plsc

B, L, D = 16, 2048, 1024
NW = 32
CHS = 32
MAXCH = 2 * B
NBUF = 2
NSL = D // 16
NGRP = 4
GSL = NSL // NGRP


def _sc_partial(x2, meta, mrow, invn):
    mesh = plsc.VectorSubcoreMesh(core_axis_name="c", subcore_axis_name="s")

    @functools.partial(
        pl.kernel,
        mesh=mesh,
        out_type=jax.ShapeDtypeStruct((2, B, D), jnp.float32),
        scratch_types=[
            pltpu.VMEM((NBUF, CHS, D), jnp.float32),
            pltpu.VMEM((B, D), jnp.float32),
            pltpu.VMEM((MAXCH, 16), jnp.int32),
            pltpu.VMEM((16,), jnp.int32),
            pltpu.VMEM((B, 16), jnp.float32),
            pltpu.VMEM((8, 128), jnp.float32),
            pltpu.VMEM((8, 128), jnp.float32),
            pltpu.VMEM_SHARED((16, B, D), jnp.float32),
            pltpu.SemaphoreType.DMA((NBUF,)),
        ],
    )
    def k(x_hbm, meta_hbm, mrow_hbm, invn_hbm, out_hbm,
          buf, acc, meta_v, m_v, invn_v, tmp, slab, shared, sem):
        c = lax.axis_index("c")
        s = lax.axis_index("s")
        wid = c * 16 + s

        pltpu.sync_copy(meta_hbm.at[wid], meta_v)
        pltpu.sync_copy(mrow_hbm.at[wid], m_v)
        pltpu.sync_copy(invn_hbm, invn_v)
        mw = m_v[...][0]

        zero = jnp.zeros((16,), jnp.float32)

        def zrow(i, carry):
            for t in range(NSL):
                acc[i, pl.ds(t * 16, 16)] = zero
            return carry

        lax.fori_loop(0, B, zrow, 0)

        def chunk_meta(j):
            v = meta_v[j, :]
            return v[0], v[1], v[2], v[3]

        def cp(j, slot):
            g0 = pl.multiple_of(chunk_meta(j)[0], 8)
            return pltpu.make_async_copy(
                x_hbm.at[pl.ds(g0, CHS), :],
                buf.at[slot],
                sem.at[slot],
            )

        for t in range(NBUF - 1):
            @pl.when(t < mw)
            def _():
                cp(t, t).start()

        def chunk_step(j, carry):
            slot = lax.rem(j, NBUF)
            jn = j + NBUF - 1

            @pl.when(jn < mw)
            def _():
                cp(jn, lax.rem(jn, NBUF)).start()

            cp(j, slot).wait()
            _, lo, hi, bat = chunk_meta(j)

            for g in range(NGRP):
                def row_step(r, part):
                    return tuple(
                        part[t] + buf[slot, r, pl.ds((g * GSL + t) * 16, 16)]
                        for t in range(GSL))

                part = lax.fori_loop(
                    lo, hi, row_step,
                    tuple(jnp.zeros((16,), jnp.float32) for _ in range(GSL)))
                for t in range(GSL):
                    sl = pl.ds((g * GSL + t) * 16, 16)
                    acc[bat, sl] = acc[bat, sl] + part[t]
            return carry

        lax.fori_loop(0, mw, chunk_step, 0)

        def scale_row(i, carry):
            iv = invn_v[i, :]
            for t in range(NSL):
                sl = pl.ds(t * 16, 16)
                acc[i, sl] = acc[i, sl] * iv
            return carry

        lax.fori_loop(0, B, scale_row, 0)

        pltpu.sync_copy(acc, shared.at[s])
        plsc.subcore_barrier()

        r0 = pl.multiple_of(lax.rem(s, 2) * 8, 8)
        c0 = pl.multiple_of(lax.div(s, 2) * 128, 128)

        for rr in range(8):
            for kk in range(8):
                slab[rr, pl.ds(kk * 16, 16)] = zero

        def src_step(src, carry):
            pltpu.sync_copy(
                shared.at[src, pl.ds(r0, 8), pl.ds(c0, 128)], tmp)
            for rr in range(8):
                for kk in range(8):
                    sl = pl.ds(kk * 16, 16)
                    slab[rr, sl] = slab[rr, sl] + tmp[rr, sl]
            return carry

        lax.fori_loop(0, 16, src_step, 0)
        pltpu.sync_copy(slab, out_hbm.at[c, pl.ds(r0, 8), pl.ds(c0, 128)])

    return k(x2, meta, mrow, invn)


def sc_mean_partials(input, length):
    n = length.astype(jnp.int32)
    x2 = input.reshape(B * L, D)
    w = jnp.arange(NW, dtype=jnp.int32)
    q = 8 * ((n + (8 * NW - 1)) // (8 * NW))
    starts = jnp.minimum(w[:, None] * q[None, :], n[None, :])
    ends = jnp.minimum((w[:, None] + 1) * q[None, :], n[None, :])
    base = jnp.arange(B, dtype=jnp.int32) * L

    def mk(cs, ce):
        g0 = base[None, :] + cs
        lo = jnp.zeros_like(cs)
        hi = ce - cs
        valid = ce > cs
        return g0, lo, hi, valid

    c0e = jnp.minimum(ends, starts + CHS)
    g0a, loa, hia, va = mk(starts, c0e)
    g0b, lob, hib, vb = mk(c0e, ends)
    batv = jnp.broadcast_to(jnp.arange(B, dtype=jnp.int32)[None, :], (NW, B))

    g0 = jnp.concatenate([g0a, g0b], axis=1)
    lo = jnp.concatenate([loa, lob], axis=1)
    hi = jnp.concatenate([hia, hib], axis=1)
    bat = jnp.concatenate([batv, batv], axis=1)
    val = jnp.concatenate([va, vb], axis=1)

    order = jnp.argsort(~val, axis=1, stable=True)
    g0 = jnp.take_along_axis(g0, order, axis=1)
    lo = jnp.take_along_axis(lo, order, axis=1)
    hi = jnp.take_along_axis(hi, order, axis=1)
    bat = jnp.take_along_axis(bat, order, axis=1)
    m = val.sum(axis=1).astype(jnp.int32)

    meta = jnp.zeros((NW, MAXCH, 16), jnp.int32)
    meta = meta.at[:, :, 0].set(g0)
    meta = meta.at[:, :, 1].set(lo)
    meta = meta.at[:, :, 2].set(hi)
    meta = meta.at[:, :, 3].set(bat)
    mrow = jnp.zeros((NW, 16), jnp.int32).at[:, 0].set(m)
    invn = jnp.broadcast_to(
        (1.0 / n.astype(jnp.float32))[:, None], (B, 16))

    return _sc_partial(x2, meta, mrow, invn)


def sc_mean(input, length):
    p = sc_mean_partials(input, length)
    return p[0] + p[1]


def kernel(input, length):
    p = sc_mean_partials(input, length)
    return p[0] + p[1]


# speedup vs baseline: 1.1034x; 1.1034x over previous
import functools
import jax
import jax.numpy as jnp
from jax import lax
from jax.experimental import pallas as pl
from jax.experimental.pallas import tpu as pltpu
from jax.experimental.pallas import tpu_sc as plsc

B, L, D = 16, 2048, 1024


CH = 128
NCH = L // CH


NW = 32
CHS = 32
MAXCH = 2 * B
NBUF = 2
NSL = D // 16
NGRP = 4
GSL = NSL // NGRP

TC_FRAC = 0.82


def _tc_body(len2_ref, in_hbm, out_ref, buf, sem):
    i = pl.program_id(0)
    n_loop = len2_ref[0, i]
    n_div = len2_ref[1, i]
    slot = lax.rem(i, 2)

    def mk(idx, sl, k):
        return pltpu.make_async_copy(
            in_hbm.at[idx, pl.ds(0, k * CH), :],
            buf.at[sl, pl.ds(0, k * CH), :],
            sem.at[sl],
        )

    def issue(idx, sl):
        kk = lax.div(len2_ref[0, idx] - 1, CH)
        lax.switch(kk, [lambda k=k: mk(idx, sl, k + 1).start()
                        for k in range(NCH)])

    def wait(idx, sl):
        kk = lax.div(len2_ref[0, idx] - 1, CH)
        lax.switch(kk, [lambda k=k: mk(idx, sl, k + 1).wait()
                        for k in range(NCH)])

    @pl.when(i == 0)
    def _():
        issue(0, 0)

    @pl.when(i + 1 < B)
    def _():
        issue(i + 1, lax.rem(i + 1, 2))

    wait(i, slot)

    nch = lax.div(n_loop - 1, CH) + 1

    def step(c, acc):
        rv = n_loop - c * CH

        def full_sum(_):
            return jnp.sum(buf[slot, pl.ds(c * CH, CH), :], axis=0)

        def masked_sum(_):
            row_id = lax.broadcasted_iota(jnp.int32, (CH, 1), 0)
            w = (row_id < rv).astype(jnp.float32)
            return jnp.sum(buf[slot, pl.ds(c * CH, CH), :] * w, axis=0)

        return acc + lax.cond(rv >= CH, full_sum, masked_sum, 0)

    acc = lax.fori_loop(0, nch, step, jnp.zeros((D,), jnp.float32))
    out_ref[i, :] = acc / n_div.astype(jnp.float32)


def _tc_partial(input, n_loop, n_div):
    len2 = jnp.stack([n_loop, n_div])
    grid_spec = pltpu.PrefetchScalarGridSpec(
        num_scalar_prefetch=1,
        grid=(B,),
        in_specs=[pl.BlockSpec(memory_space=pl.ANY)],
        out_specs=pl.BlockSpec((B, D), lambda i, len_r: (0, 0)),
        scratch_shapes=[
            pltpu.VMEM((2, L, D), jnp.float32),
            pltpu.SemaphoreType.DMA((2,)),
        ],
    )
    return pl.pallas_call(
        _tc_body,
        grid_spec=grid_spec,
        out_shape=jax.ShapeDtypeStruct((B, D), jnp.float32),
    )(len2, input)


def _sc_partial(x2, meta, mrow, invn):
    mesh = plsc.VectorSubcoreMesh(core_axis_name="c", subcore_axis_name="s")

    @functools.partial(
        pl.kernel,
        mesh=mesh,
        out_type=jax.ShapeDtypeStruct((2, B, D), jnp.float32),
        scratch_types=[
            pltpu.VMEM((NBUF, CHS, D), jnp.float32),
            pltpu.VMEM((B, D), jnp.float32),
            pltpu.VMEM((MAXCH, 16), jnp.int32),
            pltpu.VMEM((16,), jnp.int32),
            pltpu.VMEM((B, 16), jnp.float32),
            pltpu.VMEM((8, 128), jnp.float32),
            pltpu.VMEM((8, 128), jnp.float32),
            pltpu.VMEM_SHARED((16, B, D), jnp.float32),
            pltpu.SemaphoreType.DMA((NBUF,)),
        ],
    )
    def k(x_hbm, meta_hbm, mrow_hbm, invn_hbm, out_hbm,
          buf, acc, meta_v, m_v, invn_v, tmp, slab, shared, sem):
        c = lax.axis_index("c")
        s = lax.axis_index("s")
        wid = c * 16 + s

        pltpu.sync_copy(meta_hbm.at[wid], meta_v)
        pltpu.sync_copy(mrow_hbm.at[wid], m_v)
        pltpu.sync_copy(invn_hbm, invn_v)
        mw = m_v[...][0]

        zero = jnp.zeros((16,), jnp.float32)

        def zrow(i, carry):
            for t in range(NSL):
                acc[i, pl.ds(t * 16, 16)] = zero
            return carry

        lax.fori_loop(0, B, zrow, 0)

        def chunk_meta(j):
            v = meta_v[j, :]
            return v[0], v[1], v[2], v[3]

        def cp(j, slot):
            g0 = pl.multiple_of(chunk_meta(j)[0], 8)
            return pltpu.make_async_copy(
                x_hbm.at[pl.ds(g0, CHS), :],
                buf.at[slot],
                sem.at[slot],
            )

        for t in range(NBUF - 1):
            @pl.when(t < mw)
            def _():
                cp(t, t).start()

        def chunk_step(j, carry):
            slot = lax.rem(j, NBUF)
            jn = j + NBUF - 1

            @pl.when(jn < mw)
            def _():
                cp(jn, lax.rem(jn, NBUF)).start()

            cp(j, slot).wait()
            _, lo, hi, bat = chunk_meta(j)

            for g in range(NGRP):
                def row_step(r, part):
                    return tuple(
                        part[t] + buf[slot, r, pl.ds((g * GSL + t) * 16, 16)]
                        for t in range(GSL))

                part = lax.fori_loop(
                    lo, hi, row_step,
                    tuple(jnp.zeros((16,), jnp.float32) for _ in range(GSL)))
                for t in range(GSL):
                    sl = pl.ds((g * GSL + t) * 16, 16)
                    acc[bat, sl] = acc[bat, sl] + part[t]
            return carry

        lax.fori_loop(0, mw, chunk_step, 0)

        def scale_row(i, carry):
            iv = invn_v[i, :]
            for t in range(NSL):
                sl = pl.ds(t * 16, 16)
                acc[i, sl] = acc[i, sl] * iv
            return carry

        lax.fori_loop(0, B, scale_row, 0)

        pltpu.sync_copy(acc, shared.at[s])
        plsc.subcore_barrier()

        r0 = pl.multiple_of(lax.rem(s, 2) * 8, 8)
        c0 = pl.multiple_of(lax.div(s, 2) * 128, 128)

        for rr in range(8):
            for kk in range(8):
                slab[rr, pl.ds(kk * 16, 16)] = zero

        def src_step(src, carry):
            pltpu.sync_copy(
                shared.at[src, pl.ds(r0, 8), pl.ds(c0, 128)], tmp)
            for rr in range(8):
                for kk in range(8):
                    sl = pl.ds(kk * 16, 16)
                    slab[rr, sl] = slab[rr, sl] + tmp[rr, sl]
            return carry

        lax.fori_loop(0, 16, src_step, 0)
        pltpu.sync_copy(slab, out_hbm.at[c, pl.ds(r0, 8), pl.ds(c0, 128)])

    return k(x2, meta, mrow, invn)


def _sc_tail_partials(input, n, n_tc):
    x2 = input.reshape(B * L, D)
    w = jnp.arange(NW, dtype=jnp.int32)
    m = n - n_tc
    q = 8 * ((m + (8 * NW - 1)) // (8 * NW))
    starts = n_tc[None, :] + jnp.minimum(w[:, None] * q[None, :], m[None, :])
    ends = n_tc[None, :] + jnp.minimum((w[:, None] + 1) * q[None, :],
                                       m[None, :])
    base = jnp.arange(B, dtype=jnp.int32) * L

    def mk(cs, ce):
        dma0 = jnp.minimum(cs, L - CHS)
        g0 = base[None, :] + dma0
        lo = cs - dma0
        hi = ce - dma0
        valid = ce > cs
        return g0, lo, hi, valid

    c0e = jnp.minimum(ends, starts + CHS)
    g0a, loa, hia, va = mk(starts, c0e)
    g0b, lob, hib, vb = mk(c0e, ends)
    batv = jnp.broadcast_to(jnp.arange(B, dtype=jnp.int32)[None, :], (NW, B))

    g0 = jnp.concatenate([g0a, g0b], axis=1)
    lo = jnp.concatenate([loa, lob], axis=1)
    hi = jnp.concatenate([hia, hib], axis=1)
    bat = jnp.concatenate([batv, batv], axis=1)
    val = jnp.concatenate([va, vb], axis=1)

    order = jnp.argsort(~val, axis=1, stable=True)
    g0 = jnp.take_along_axis(g0, order, axis=1)
    lo = jnp.take_along_axis(lo, order, axis=1)
    hi = jnp.take_along_axis(hi, order, axis=1)
    bat = jnp.take_along_axis(bat, order, axis=1)
    mcnt = val.sum(axis=1).astype(jnp.int32)

    meta = jnp.zeros((NW, MAXCH, 16), jnp.int32)
    meta = meta.at[:, :, 0].set(g0)
    meta = meta.at[:, :, 1].set(lo)
    meta = meta.at[:, :, 2].set(hi)
    meta = meta.at[:, :, 3].set(bat)
    mrow = jnp.zeros((NW, 16), jnp.int32).at[:, 0].set(mcnt)
    invn = jnp.broadcast_to(
        (1.0 / n.astype(jnp.float32))[:, None], (B, 16))

    return _sc_partial(x2, meta, mrow, invn)


def kernel(input, length):
    n = length.astype(jnp.int32)
    n_tc = jnp.clip(8 * jnp.int32(jnp.round(n * (TC_FRAC / 8.0))), 8, n)
    tc = _tc_partial(input, n_tc, n)
    p = _sc_tail_partials(input, n, n_tc)
    return tc + p[0] + p[1]


# speedup vs baseline: 2.2215x; 2.0134x over previous
import functools
import jax
import jax.numpy as jnp
from jax import lax
from jax.experimental import pallas as pl
from jax.experimental.pallas import tpu as pltpu
from jax.experimental.pallas import tpu_sc as plsc

B, L, D = 16, 2048, 1024


CH = 128
NCH = L // CH


NW = 32
CHS = 32
MAXCH = 16
NBUF = 2
NSL = D // 16
NGRP = 4
GSL = NSL // NGRP

TC_FRAC = 0.82


def _tc_body(len2_ref, in_hbm, out_ref, buf, sem):
    i = pl.program_id(0)
    n_loop = len2_ref[0, i]
    n_div = len2_ref[1, i]
    slot = lax.rem(i, 2)

    def mk(idx, sl, k):
        return pltpu.make_async_copy(
            in_hbm.at[idx, pl.ds(0, k * CH), :],
            buf.at[sl, pl.ds(0, k * CH), :],
            sem.at[sl],
        )

    def issue(idx, sl):
        kk = lax.div(len2_ref[0, idx] - 1, CH)
        lax.switch(kk, [lambda k=k: mk(idx, sl, k + 1).start()
                        for k in range(NCH)])

    def wait(idx, sl):
        kk = lax.div(len2_ref[0, idx] - 1, CH)
        lax.switch(kk, [lambda k=k: mk(idx, sl, k + 1).wait()
                        for k in range(NCH)])

    @pl.when(i == 0)
    def _():
        issue(0, 0)

    @pl.when(i + 1 < B)
    def _():
        issue(i + 1, lax.rem(i + 1, 2))

    wait(i, slot)

    nch = lax.div(n_loop - 1, CH) + 1

    def step(c, acc):
        rv = n_loop - c * CH

        def full_sum(_):
            return jnp.sum(buf[slot, pl.ds(c * CH, CH), :], axis=0)

        def masked_sum(_):
            row_id = lax.broadcasted_iota(jnp.int32, (CH, 1), 0)
            w = (row_id < rv).astype(jnp.float32)
            return jnp.sum(buf[slot, pl.ds(c * CH, CH), :] * w, axis=0)

        return acc + lax.cond(rv >= CH, full_sum, masked_sum, 0)

    acc = lax.fori_loop(0, nch, step, jnp.zeros((D,), jnp.float32))
    out_ref[i, :] = acc / n_div.astype(jnp.float32)


def _tc_partial(input, n_loop, n_div):
    len2 = jnp.stack([n_loop, n_div])
    grid_spec = pltpu.PrefetchScalarGridSpec(
        num_scalar_prefetch=1,
        grid=(B,),
        in_specs=[pl.BlockSpec(memory_space=pl.ANY)],
        out_specs=pl.BlockSpec((B, D), lambda i, len_r: (0, 0)),
        scratch_shapes=[
            pltpu.VMEM((2, L, D), jnp.float32),
            pltpu.SemaphoreType.DMA((2,)),
        ],
    )
    return pl.pallas_call(
        _tc_body,
        grid_spec=grid_spec,
        out_shape=jax.ShapeDtypeStruct((B, D), jnp.float32),
    )(len2, input)


def _sc_partial(x2, meta, mrow, invn):
    mesh = plsc.VectorSubcoreMesh(core_axis_name="c", subcore_axis_name="s")

    @functools.partial(
        pl.kernel,
        mesh=mesh,
        out_type=jax.ShapeDtypeStruct((2, B, D), jnp.float32),
        scratch_types=[
            pltpu.VMEM((NBUF, CHS, D), jnp.float32),
            pltpu.VMEM((B, D), jnp.float32),
            pltpu.VMEM((MAXCH, 16), jnp.int32),
            pltpu.VMEM((16,), jnp.int32),
            pltpu.VMEM((B, 16), jnp.float32),
            pltpu.VMEM((8, 128), jnp.float32),
            pltpu.VMEM((8, 128), jnp.float32),
            pltpu.VMEM_SHARED((16, B, D), jnp.float32),
            pltpu.SemaphoreType.DMA((NBUF,)),
        ],
    )
    def k(x_hbm, meta_hbm, mrow_hbm, invn_hbm, out_hbm,
          buf, acc, meta_v, m_v, invn_v, tmp, slab, shared, sem):
        c = lax.axis_index("c")
        s = lax.axis_index("s")
        wid = c * 16 + s

        pltpu.sync_copy(meta_hbm.at[wid], meta_v)
        pltpu.sync_copy(mrow_hbm.at[wid], m_v)
        pltpu.sync_copy(invn_hbm, invn_v)
        mw = m_v[...][0]

        zero = jnp.zeros((16,), jnp.float32)

        def zrow(i, carry):
            for t in range(NSL):
                acc[i, pl.ds(t * 16, 16)] = zero
            return carry

        lax.fori_loop(0, B, zrow, 0)

        def chunk_meta(j):
            v = meta_v[j, :]
            return v[0], v[1], v[2], v[3]

        def cp(j, slot):
            g0 = pl.multiple_of(chunk_meta(j)[0], 8)
            return pltpu.make_async_copy(
                x_hbm.at[pl.ds(g0, CHS), :],
                buf.at[slot],
                sem.at[slot],
            )

        for t in range(NBUF - 1):
            @pl.when(t < mw)
            def _():
                cp(t, t).start()

        def chunk_step(j, carry):
            slot = lax.rem(j, NBUF)
            jn = j + NBUF - 1

            @pl.when(jn < mw)
            def _():
                cp(jn, lax.rem(jn, NBUF)).start()

            cp(j, slot).wait()
            _, lo, hi, bat = chunk_meta(j)

            for g in range(NGRP):
                def row_step(r, part):
                    return tuple(
                        part[t] + buf[slot, r, pl.ds((g * GSL + t) * 16, 16)]
                        for t in range(GSL))

                part = lax.fori_loop(
                    lo, hi, row_step,
                    tuple(jnp.zeros((16,), jnp.float32) for _ in range(GSL)))
                for t in range(GSL):
                    sl = pl.ds((g * GSL + t) * 16, 16)
                    acc[bat, sl] = acc[bat, sl] + part[t]
            return carry

        lax.fori_loop(0, mw, chunk_step, 0)

        def scale_row(i, carry):
            iv = invn_v[i, :]
            for t in range(NSL):
                sl = pl.ds(t * 16, 16)
                acc[i, sl] = acc[i, sl] * iv
            return carry

        lax.fori_loop(0, B, scale_row, 0)

        pltpu.sync_copy(acc, shared.at[s])
        plsc.subcore_barrier()

        r0 = pl.multiple_of(lax.rem(s, 2) * 8, 8)
        c0 = pl.multiple_of(lax.div(s, 2) * 128, 128)

        for rr in range(8):
            for kk in range(8):
                slab[rr, pl.ds(kk * 16, 16)] = zero

        def src_step(src, carry):
            pltpu.sync_copy(
                shared.at[src, pl.ds(r0, 8), pl.ds(c0, 128)], tmp)
            for rr in range(8):
                for kk in range(8):
                    sl = pl.ds(kk * 16, 16)
                    slab[rr, sl] = slab[rr, sl] + tmp[rr, sl]
            return carry

        lax.fori_loop(0, 16, src_step, 0)
        pltpu.sync_copy(slab, out_hbm.at[c, pl.ds(r0, 8), pl.ds(c0, 128)])

    return k(x2, meta, mrow, invn)


def _sc_tail_partials(input, n, n_tc):
    x2 = input.reshape(B * L, D)
    m = n - n_tc
    q2 = 8 * ((m + 15) // 16)
    wi = jnp.arange(NW, dtype=jnp.int32)
    bat1 = wi // 2
    h = wi % 2
    ntb = n_tc[bat1]
    mb = m[bat1]
    q2b = q2[bat1]
    ws = ntb + jnp.minimum(h * q2b, mb)
    we = ntb + jnp.minimum((h + 1) * q2b, mb)

    k = jnp.arange(MAXCH, dtype=jnp.int32)
    cs = ws[:, None] + CHS * k[None, :]
    ce = jnp.minimum(we[:, None], cs + CHS)
    valid = ce > cs
    dma0 = jnp.minimum(cs, L - CHS)
    g0 = (bat1 * L)[:, None] + dma0
    lo = cs - dma0
    hi = jnp.where(valid, ce - dma0, lo)
    bat = jnp.broadcast_to(bat1[:, None], (NW, MAXCH))
    mcnt = valid.sum(axis=1).astype(jnp.int32)

    meta = jnp.zeros((NW, MAXCH, 16), jnp.int32)
    meta = meta.at[:, :, 0].set(g0)
    meta = meta.at[:, :, 1].set(lo)
    meta = meta.at[:, :, 2].set(hi)
    meta = meta.at[:, :, 3].set(bat)
    mrow = jnp.zeros((NW, 16), jnp.int32).at[:, 0].set(mcnt)
    invn = jnp.broadcast_to(
        (1.0 / n.astype(jnp.float32))[:, None], (B, 16))

    return _sc_partial(x2, meta, mrow, invn)


def kernel(input, length):
    n = length.astype(jnp.int32)
    n_tc = jnp.clip(8 * jnp.int32(jnp.round(n * (TC_FRAC / 8.0))), 8, n)
    tc = _tc_partial(input, n_tc, n)
    p = _sc_tail_partials(input, n, n_tc)
    return tc + p[0] + p[1]
